# baseline (speedup 1.0000x reference)
import functools

import jax
import jax.numpy as jnp
from jax import lax
from jax.experimental import pallas as pl
from jax.experimental.pallas import tpu as pltpu
from jax.experimental.pallas import tpu_sc as plsc

H = 4096
W = 8192
B = 1048576
NC = 2
NS = 16
NW = NC * NS
N_PER_W = B // NW
CHUNK = 8192
NCHUNK = N_PER_W // CHUNK
LANES = 16


def _round_half_even(v):
    t0 = v.astype(jnp.int32)
    frac = v - t0.astype(jnp.float32)
    half = jnp.float32(0.5)
    up = (frac > half) | ((frac == half) & ((t0 & 1) == 1))
    return t0 + up.astype(jnp.int32)


def _make_kernel():
    mesh = plsc.VectorSubcoreMesh(core_axis_name="c", subcore_axis_name="s")

    @functools.partial(
        pl.kernel,
        mesh=mesh,
        out_type=jax.ShapeDtypeStruct((B,), jnp.float32),
        scratch_types=[
            pltpu.VMEM((2 * CHUNK,), jnp.float32),
            pltpu.VMEM((CHUNK,), jnp.int32),
            pltpu.VMEM((CHUNK,), jnp.float32),
            pltpu.SemaphoreType.DMA,
        ],
        compiler_params=pltpu.CompilerParams(needs_layout_passes=False),
    )
    def night_light(x_hbm, f_hbm, y_hbm, xv, idxv, yv, sem):
        wid = lax.axis_index("s") * NC + lax.axis_index("c")
        base0 = wid * N_PER_W
        evens = lax.iota(jnp.int32, LANES) * 2

        def chunk_body(k, carry):
            base = base0 + k * CHUNK
            pltpu.sync_copy(x_hbm.at[pl.ds(2 * base, 2 * CHUNK)], xv)

            def pt_body(i, c2):
                lane = evens + i * (2 * LANES)
                x0 = plsc.load_gather(xv, [lane])
                x1 = plsc.load_gather(xv, [lane + 1])
                cn = jnp.clip((x0 + 1.0) * jnp.float32(W // 2),
                              jnp.float32(0.0), jnp.float32(W - 1))
                rn = jnp.clip((x1 + 1.0) * jnp.float32(H // 2),
                              jnp.float32(0.0), jnp.float32(H - 1))
                col = _round_half_even(cn)
                row = _round_half_even(rn)
                idxv[pl.ds(i * LANES, LANES)] = (row << 13) | col
                return c2

            lax.fori_loop(0, CHUNK // LANES, pt_body, 0)
            pltpu.async_copy(f_hbm.at[idxv], yv, sem).wait()
            pltpu.sync_copy(yv, y_hbm.at[pl.ds(base, CHUNK)])
            return carry

        lax.fori_loop(0, NCHUNK, chunk_body, 0)

    return night_light


_night_light = _make_kernel()


@jax.jit
def kernel(x, f):
    return _night_light(x.reshape(-1), f.reshape(-1))


# speedup vs baseline: 14.8139x; 14.8139x over previous
import functools

import jax
import jax.numpy as jnp
from jax import lax
from jax.experimental import pallas as pl
from jax.experimental.pallas import tpu as pltpu
from jax.experimental.pallas import tpu_sc as plsc

H = 4096
W = 8192
B = 1048576
NC = 2
NS = 16
NW = NC * NS
N_PER_W = B // NW
CHUNK = 8192
NCHUNK = N_PER_W // CHUNK
LANES = 16


def _round_half_even(v):
    t0 = v.astype(jnp.int32)
    frac = v - t0.astype(jnp.float32)
    half = jnp.float32(0.5)
    up = (frac > half) | ((frac == half) & ((t0 & 1) == 1))
    return t0 + up.astype(jnp.int32)


def _make_kernel():
    mesh = plsc.VectorSubcoreMesh(core_axis_name="c", subcore_axis_name="s")

    @functools.partial(
        pl.kernel,
        mesh=mesh,
        out_type=jax.ShapeDtypeStruct((B,), jnp.float32),
        scratch_types=[
            pltpu.VMEM((2 * CHUNK,), jnp.float32),
            pltpu.VMEM((CHUNK,), jnp.int32),
            pltpu.VMEM((CHUNK,), jnp.float32),
            pltpu.SemaphoreType.DMA,
        ],
        compiler_params=pltpu.CompilerParams(needs_layout_passes=False),
    )
    def night_light(x_hbm, f_hbm, y_hbm, xv, idxv, yv, sem):
        wid = lax.axis_index("s") * NC + lax.axis_index("c")
        base0 = wid * N_PER_W

        def chunk_body(k, carry):
            base = base0 + k * CHUNK
            pltpu.sync_copy(x_hbm.at[pl.ds(2 * base, 2 * CHUNK)], xv)

            def pt_body(i, c2):
                off = (i >> 3) * 256 + (i & 7) * LANES
                x0 = xv[pl.ds(off, LANES)]
                x1 = xv[pl.ds(off + 128, LANES)]
                cn = jnp.clip((x0 + 1.0) * jnp.float32(W // 2),
                              jnp.float32(0.0), jnp.float32(W - 1))
                rn = jnp.clip((x1 + 1.0) * jnp.float32(H // 2),
                              jnp.float32(0.0), jnp.float32(H - 1))
                col = _round_half_even(cn)
                row = _round_half_even(rn)
                tile = ((row >> 3) << 6) | (col >> 7)
                within = ((row & 7) << 7) | (col & 127)
                idxv[pl.ds(i * LANES, LANES)] = (tile << 10) | within
                return c2

            lax.fori_loop(0, CHUNK // LANES, pt_body, 0)
            pltpu.async_copy(f_hbm.at[idxv], yv, sem).wait()
            pltpu.sync_copy(yv, y_hbm.at[pl.ds(base, CHUNK)])
            return carry

        lax.fori_loop(0, NCHUNK, chunk_body, 0)

    return night_light


_night_light = _make_kernel()


@jax.jit
def kernel(x, f):
    x_phys = x.reshape(B // 128, 128, 2).transpose(0, 2, 1).reshape(2 * B)
    f_phys = (
        f.reshape(H // 8, 8, W // 128, 128)
        .transpose(0, 2, 1, 3)
        .reshape(H * W)
    )
    return _night_light(x_phys, f_phys)


# speedup vs baseline: 21.1718x; 1.4292x over previous
import functools

import jax
import jax.numpy as jnp
from jax import lax
from jax.experimental import pallas as pl
from jax.experimental.pallas import tpu as pltpu
from jax.experimental.pallas import tpu_sc as plsc

H = 4096
W = 8192
B = 1048576
NC = 2
NS = 16
NW = NC * NS
N_PER_W = B // NW
CHUNK = 8192
NCHUNK = N_PER_W // CHUNK
LANES = 16


def _round_half_even(v):
    t0 = v.astype(jnp.int32)
    frac = v - t0.astype(jnp.float32)
    half = jnp.float32(0.5)
    up = (frac > half) | ((frac == half) & ((t0 & 1) == 1))
    return t0 + up.astype(jnp.int32)


def _make_kernel():
    mesh = plsc.VectorSubcoreMesh(core_axis_name="c", subcore_axis_name="s")

    @functools.partial(
        pl.kernel,
        mesh=mesh,
        out_type=jax.ShapeDtypeStruct((B,), jnp.float32),
        scratch_types=[
            pltpu.VMEM((2 * CHUNK,), jnp.float32),
            pltpu.VMEM((2 * CHUNK,), jnp.float32),
            pltpu.VMEM((CHUNK,), jnp.int32),
            pltpu.VMEM((CHUNK,), jnp.int32),
            pltpu.VMEM((CHUNK,), jnp.float32),
            pltpu.VMEM((CHUNK,), jnp.float32),
            pltpu.SemaphoreType.DMA,
            pltpu.SemaphoreType.DMA,
            pltpu.SemaphoreType.DMA,
            pltpu.SemaphoreType.DMA,
            pltpu.SemaphoreType.DMA,
            pltpu.SemaphoreType.DMA,
        ],
        compiler_params=pltpu.CompilerParams(needs_layout_passes=False),
    )
    def night_light(x_hbm, f_hbm, y_hbm,
                    xv0, xv1, ix0, ix1, yv0, yv1,
                    sx0, sx1, sg0, sg1, sw0, sw1):
        wid = lax.axis_index("s") * NC + lax.axis_index("c")
        base0 = wid * N_PER_W
        xv, ix, yv = [xv0, xv1], [ix0, ix1], [yv0, yv1]
        sx, sg, sw = [sx0, sx1], [sg0, sg1], [sw0, sw1]

        def xcopy(k):
            base = base0 + k * CHUNK
            return pltpu.async_copy(
                x_hbm.at[pl.ds(2 * base, 2 * CHUNK)], xv[k % 2], sx[k % 2])

        def compute(k):
            b = k % 2

            def pt_body(i, c2):
                off = (i >> 3) * 256 + (i & 7) * LANES
                x0 = xv[b][pl.ds(off, LANES)]
                x1 = xv[b][pl.ds(off + 128, LANES)]
                cn = jnp.clip((x0 + 1.0) * jnp.float32(W // 2),
                              jnp.float32(0.0), jnp.float32(W - 1))
                rn = jnp.clip((x1 + 1.0) * jnp.float32(H // 2),
                              jnp.float32(0.0), jnp.float32(H - 1))
                col = _round_half_even(cn)
                row = _round_half_even(rn)
                tile = ((row >> 3) << 6) | (col >> 7)
                within = ((row & 7) << 7) | (col & 127)
                ix[b][pl.ds(i * LANES, LANES)] = (tile << 10) | within
                return c2

            lax.fori_loop(0, CHUNK // LANES, pt_body, 0, unroll=2)

        def gather(k):
            return pltpu.async_copy(f_hbm.at[ix[k % 2]], yv[k % 2], sg[k % 2])

        def wb(k):
            base = base0 + k * CHUNK
            return pltpu.async_copy(
                yv[k % 2], y_hbm.at[pl.ds(base, CHUNK)], sw[k % 2])

        hx, hg, hw = {}, {}, {}
        hx[0] = xcopy(0)
        for k in range(NCHUNK):
            hx[k].wait()
            if k + 1 < NCHUNK:
                hx[k + 1] = xcopy(k + 1)
            if k >= 2:
                hw[k - 2].wait()
            compute(k)
            if k >= 1:
                hg[k - 1].wait()
                hw[k - 1] = wb(k - 1)
            hg[k] = gather(k)
        hg[NCHUNK - 1].wait()
        hw[NCHUNK - 1] = wb(NCHUNK - 1)
        hw[NCHUNK - 2].wait()
        hw[NCHUNK - 1].wait()

    return night_light


_night_light = _make_kernel()


@jax.jit
def kernel(x, f):
    x_phys = x.reshape(B // 128, 128, 2).transpose(0, 2, 1).reshape(2 * B)
    f_phys = (
        f.reshape(H // 8, 8, W // 128, 128)
        .transpose(0, 2, 1, 3)
        .reshape(H * W)
    )
    return _night_light(x_phys, f_phys)
